# baseline (speedup 1.0000x reference)
import functools

import jax
import jax.numpy as jnp
import numpy as np
from jax import lax
from jax.experimental import pallas as pl
from jax.experimental.pallas import tpu as pltpu
from jax.experimental.pallas import tpu_sc as plsc

H = 64
NUM_RBF = 32
CUT_HI = 4.5
NUM_LAYERS = 2
PI = float(np.pi)


def _silu(x):
    return x * jax.nn.sigmoid(x)


def _cutoff(d):
    return 0.5 * (jnp.cos(d * (PI / CUT_HI)) + 1.0) * (d < CUT_HI)


def _expnorm_rbf(d):
    alpha = 5.0 / CUT_HI
    start = float(np.exp(-CUT_HI))
    means = jnp.linspace(start, 1.0, NUM_RBF).astype(jnp.float32)
    betas = (2.0 / NUM_RBF * (1.0 - start)) ** -2
    d = d[:, None]
    return _cutoff(d) * jnp.exp(-betas * (jnp.exp(-alpha * d) - means) ** 2)


def _layer_norm(x, g, b):
    m = x.mean(-1, keepdims=True)
    v = ((x - m) ** 2).mean(-1, keepdims=True)
    return (x - m) / jnp.sqrt(v + 1e-5) * g + b


def _comps_norm(c):
    i = c[..., 0]
    a2 = c[..., 1] ** 2 + c[..., 2] ** 2 + c[..., 3] ** 2
    s00, s01, s02, s11, s12 = c[..., 4], c[..., 5], c[..., 6], c[..., 7], c[..., 8]
    s22 = -(s00 + s11)
    sn = s00**2 + s11**2 + s22**2 + 2.0 * (s01**2 + s02**2 + s12**2)
    return 3.0 * i**2 + 2.0 * a2 + sn


def _comps_to_full(c):
    i = c[..., 0]
    a0, a1, a2 = c[..., 1], c[..., 2], c[..., 3]
    s00, s01, s02, s11, s12 = c[..., 4], c[..., 5], c[..., 6], c[..., 7], c[..., 8]
    s22 = -(s00 + s11)
    r0 = jnp.stack([i + s00, s01 - a2, s02 + a1], -1)
    r1 = jnp.stack([s01 + a2, i + s11, s12 - a0], -1)
    r2 = jnp.stack([s02 - a1, s12 + a0, i + s22], -1)
    return jnp.stack([r0, r1, r2], -2)


def _full_to_comps(T):
    i = (T[..., 0, 0] + T[..., 1, 1] + T[..., 2, 2]) / 3.0
    a0 = 0.5 * (T[..., 2, 1] - T[..., 1, 2])
    a1 = 0.5 * (T[..., 0, 2] - T[..., 2, 0])
    a2 = 0.5 * (T[..., 1, 0] - T[..., 0, 1])
    s00 = T[..., 0, 0] - i
    s01 = 0.5 * (T[..., 0, 1] + T[..., 1, 0])
    s02 = 0.5 * (T[..., 0, 2] + T[..., 2, 0])
    s11 = T[..., 1, 1] - i
    s12 = 0.5 * (T[..., 1, 2] + T[..., 2, 1])
    return jnp.stack([i, a0, a1, a2, s00, s01, s02, s11, s12], -1)


_G = np.array([0, 1, 1, 1, 2, 2, 2, 2, 2])


NC_T = 64
CHUNKS = 160
CPT = CHUNKS // 32
NPAD = NC_T * CHUNKS
EB = 64
ROW = 9 * H
ROWP = 640
CW = 256
_G9 = (0, 1, 1, 1, 2, 2, 2, 2, 2)


def _lane_select(vec, lane, fill):
    sel = lax.iota(jnp.int32, 16) == jnp.full((16,), lane, jnp.int32)
    return jnp.max(jnp.where(sel, vec, jnp.full((16,), fill, vec.dtype)))


def _chunk_bounds(bnd_hbm, bndb, ch):
    sl0 = pl.multiple_of(jnp.bitwise_and(ch, jnp.int32(-8)), 8)
    pltpu.sync_copy(bnd_hbm.at[pl.ds(sl0, 16)], bndb)
    lane = ch - sl0
    s_true = _lane_select(bndb[pl.ds(0, 16)], lane, 0)
    e_true = _lane_select(bndb[pl.ds(0, 16)], lane + 1, 0)
    s_al = jnp.bitwise_and(s_true, jnp.int32(-8))
    nb = lax.div(e_true - s_al + jnp.int32(EB - 1), jnp.int32(EB))
    return s_true, e_true, s_al, nb


def _zero_acc(acc):
    zf = jnp.zeros((16,), jnp.float32)

    def row_body(r, _):
        for v in range(ROWP // 16):
            acc[r, pl.ds(v * 16, 16)] = zf
        return 0

    lax.fori_loop(0, NC_T, row_body, 0)


def _valid_f(gid, s_true, e_true):
    ok = jnp.logical_and(gid >= s_true, gid < e_true)
    okv = jnp.full((16,), ok)
    return jnp.where(okv, jnp.full((16,), 1.0, jnp.float32),
                     jnp.zeros((16,), jnp.float32))


def _local_row(ldstb, j):
    base = pl.multiple_of(jnp.bitwise_and(j, jnp.int32(-16)), 16)
    return _lane_select(ldstb[pl.ds(base, 16)], jnp.bitwise_and(j, 15), 0)


def _mp_inter_body(T_hbm, srcl_hbm, ldst_hbm, coef_hbm, bnd_hbm, out_hbm,
                   acc, idxb, ldstb, coefb, rowsb, bndb, sem):
    w = lax.axis_index("s") * 2 + lax.axis_index("c")
    for q in range(CPT):
        ch = w * CPT + q
        _zero_acc(acc)
        s_true, e_true, s_al, nb = _chunk_bounds(bnd_hbm, bndb, ch)

        def batch_body(i, _):
            sb = pl.multiple_of(s_al + i * EB, 8)
            pltpu.sync_copy(srcl_hbm.at[pl.ds(sb, EB)], idxb)
            pltpu.sync_copy(ldst_hbm.at[pl.ds(sb, EB)], ldstb)
            pltpu.sync_copy(coef_hbm.at[pl.ds(sb, EB)], coefb)
            pltpu.async_copy(T_hbm.at[idxb], rowsb, sem).wait()

            def edge_body(j, _2):
                okf = _valid_f(sb + j, s_true, e_true)
                ld = _local_row(ldstb, j)
                cvs = [coefb[j, pl.ds(wi * 16, 16)] * okf for wi in range(12)]
                for v in range(36):
                    g = _G9[v // 4]
                    sl = pl.ds(v * 16, 16)
                    acc[ld, sl] = acc[ld, sl] + rowsb[j, sl] * cvs[g * 4 + (v % 4)]
                return 0

            lax.fori_loop(0, EB, edge_body, 0)
            return 0

        lax.fori_loop(0, nb, batch_body, 0)
        pltpu.sync_copy(acc, out_hbm.at[pl.ds(ch * NC_T, NC_T)])


def _mp_embed_body(pz_hbm, dstg_hbm, srcl_hbm, ldst_hbm, w_hbm, bnd_hbm,
                   out_hbm,
                   acc, dgb, idxb, ldstb, pab, pbb, wb, bndb, sem):
    w = lax.axis_index("s") * 2 + lax.axis_index("c")
    for q in range(CPT):
        ch = w * CPT + q
        _zero_acc(acc)
        s_true, e_true, s_al, nb = _chunk_bounds(bnd_hbm, bndb, ch)

        def batch_body(i, _):
            sb = pl.multiple_of(s_al + i * EB, 8)
            pltpu.sync_copy(dstg_hbm.at[pl.ds(sb, EB)], dgb)
            pltpu.sync_copy(srcl_hbm.at[pl.ds(sb, EB)], idxb)
            pltpu.sync_copy(ldst_hbm.at[pl.ds(sb, EB)], ldstb)
            pltpu.sync_copy(w_hbm.at[pl.ds(sb, EB)], wb)
            cp_a = pltpu.async_copy(pz_hbm.at[dgb], pab, sem)
            cp_b = pltpu.async_copy(pz_hbm.at[idxb], pbb, sem)
            cp_a.wait()
            cp_b.wait()

            def edge_body(j, _2):
                okf = _valid_f(sb + j, s_true, e_true)
                ld = _local_row(ldstb, j)
                zv = [pab[j, pl.ds(wi * 16, 16)] + pbb[j, pl.ds(64 + wi * 16, 16)]
                      for wi in range(4)]
                cvs = [zv[wi % 4] * wb[j, pl.ds(wi * 16, 16)] * okf
                       for wi in range(12)]
                gvec = wb[j, pl.ds(192, 16)]
                gvs = [_lane_select(gvec, cc, jnp.float32(-3e38))
                       for cc in range(9)]
                for v in range(36):
                    cc = v // 4
                    g = _G9[cc]
                    sl = pl.ds(v * 16, 16)
                    acc[ld, sl] = acc[ld, sl] + cvs[g * 4 + (v % 4)] * gvs[cc]
                return 0

            lax.fori_loop(0, EB, edge_body, 0)
            return 0

        lax.fori_loop(0, nb, batch_body, 0)
        pltpu.sync_copy(acc, out_hbm.at[pl.ds(ch * NC_T, NC_T)])


_SC_MESH = dict(
    mesh=plsc.VectorSubcoreMesh(core_axis_name="c", subcore_axis_name="s"),
    compiler_params=pltpu.CompilerParams(needs_layout_passes=False),
)


def _mp_inter_sc(T9, srcl, ldst, coef, bnd):
    f = pl.kernel(
        _mp_inter_body,
        out_type=jax.ShapeDtypeStruct((NPAD, ROWP), jnp.float32),
        scratch_types=[
            pltpu.VMEM((NC_T, ROWP), jnp.float32),
            pltpu.VMEM((EB,), jnp.int32),
            pltpu.VMEM((EB,), jnp.int32),
            pltpu.VMEM((EB, CW), jnp.float32),
            pltpu.VMEM((EB, ROWP), jnp.float32),
            pltpu.VMEM((16,), jnp.int32),
            pltpu.SemaphoreType.DMA,
        ],
        **_SC_MESH,
    )
    return f(T9, srcl, ldst, coef, bnd)


def _mp_embed_sc(pz, dstg, srcl, ldst, w123, bnd):
    f = pl.kernel(
        _mp_embed_body,
        out_type=jax.ShapeDtypeStruct((NPAD, ROWP), jnp.float32),
        scratch_types=[
            pltpu.VMEM((NC_T, ROWP), jnp.float32),
            pltpu.VMEM((EB,), jnp.int32),
            pltpu.VMEM((EB,), jnp.int32),
            pltpu.VMEM((EB,), jnp.int32),
            pltpu.VMEM((EB, 128), jnp.float32),
            pltpu.VMEM((EB, 128), jnp.float32),
            pltpu.VMEM((EB, CW), jnp.float32),
            pltpu.VMEM((16,), jnp.int32),
            pltpu.SemaphoreType.DMA,
        ],
        **_SC_MESH,
    )
    return f(pz, dstg, srcl, ldst, w123, bnd)


_NB = 512


def _head_body(x_ref, g_ref, b_ref, w_ref, lb_ref, o_ref):
    x = x_ref[...]
    i = x[:, 0:64]
    a0, a1, a2 = x[:, 64:128], x[:, 128:192], x[:, 192:256]
    s00, s01, s02 = x[:, 256:320], x[:, 320:384], x[:, 384:448]
    s11, s12 = x[:, 448:512], x[:, 512:576]
    s22 = -(s00 + s11)
    nI = 3.0 * i * i
    nA = 2.0 * (a0 * a0 + a1 * a1 + a2 * a2)
    nS = s00 * s00 + s11 * s11 + s22 * s22 + 2.0 * (s01 * s01 + s02 * s02 + s12 * s12)
    v = jnp.concatenate([nI, nA, nS], axis=1)
    m = v.mean(-1, keepdims=True)
    var = ((v - m) ** 2).mean(-1, keepdims=True)
    v = (v - m) / jnp.sqrt(var + 1e-5) * g_ref[...] + b_ref[...]
    y = jnp.dot(v, w_ref[...], preferred_element_type=jnp.float32) + lb_ref[...]
    o_ref[...] = y * jax.nn.sigmoid(y)


def _head(Xc, g, b, w, lb):
    n = Xc.shape[0]
    npad = ((n + _NB - 1) // _NB) * _NB
    x = jnp.swapaxes(Xc, 1, 2).reshape(n, 9 * H)
    if npad != n:
        x = jnp.pad(x, ((0, npad - n), (0, 0)))
    out = pl.pallas_call(
        _head_body,
        grid=(npad // _NB,),
        in_specs=[
            pl.BlockSpec((_NB, 9 * H), lambda i: (i, 0)),
            pl.BlockSpec((3 * H,), lambda i: (0,)),
            pl.BlockSpec((3 * H,), lambda i: (0,)),
            pl.BlockSpec((3 * H, H), lambda i: (0, 0)),
            pl.BlockSpec((H,), lambda i: (0,)),
        ],
        out_specs=pl.BlockSpec((_NB, H), lambda i: (i, 0)),
        out_shape=jax.ShapeDtypeStruct((npad, H), jnp.float32),
    )(x, g, b, w, lb)
    return out[:n]


def kernel(z, edge_index, edge_weight, edge_vec, params):
    p = params
    n = z.shape[0]
    E = edge_index.shape[1]
    EPAD = E + 2 * EB

    dst0 = edge_index[0].astype(jnp.int32)
    order = jnp.argsort(dst0)
    dst = dst0[order]
    src = edge_index[1].astype(jnp.int32)[order]
    ew = edge_weight[order]
    ev = edge_vec[order]

    b = jnp.searchsorted(dst, jnp.arange(0, NPAD + 1, NC_T,
                                         dtype=jnp.int32)).astype(jnp.int32)
    bnd = jnp.pad(b, (0, 176 - (CHUNKS + 1)), mode='edge')
    ldst = jnp.bitwise_and(dst, NC_T - 1)
    epad = EPAD - E
    srcl = jnp.pad(src, (0, epad))
    dstg = jnp.pad(dst, (0, epad))
    ldst = jnp.pad(ldst, (0, epad))

    edge_attr = _expnorm_rbf(ew)
    C = _cutoff(ew)
    evn = ev / ew[:, None]

    Wd = jnp.concatenate([p['dproj_w'][0], p['dproj_w'][1],
                          p['dproj_w'][2]], axis=1)
    bd = jnp.concatenate([p['dproj_b'][0], p['dproj_b'][1], p['dproj_b'][2]])
    W123 = (edge_attr @ Wd + bd) * C[:, None]
    Z = jnp.take(p['emb'], z, axis=0)
    PZ = jnp.concatenate([Z @ p['emb2_w'][:H] + p['emb2_b'],
                          Z @ p['emb2_w'][H:]], axis=1)
    v0, v1, v2 = evn[:, 0], evn[:, 1], evn[:, 2]
    q = (v0 * v0 + v1 * v1 + v2 * v2) / 3.0
    gc = jnp.stack([jnp.ones_like(v0), v0, v1, v2,
                    v0 * v0 - q, v0 * v1, v0 * v2, v1 * v1 - q, v1 * v2], -1)
    Wcat = jnp.pad(jnp.concatenate([W123, gc], axis=1),
                   ((0, epad), (0, CW - 192 - 9)))
    acc = _mp_embed_sc(PZ, dstg, srcl, ldst, Wcat, bnd)
    Xc = jnp.swapaxes(acc[:n, :ROW].reshape(n, 9, H), 1, 2)

    norm = _layer_norm(_comps_norm(Xc), p['te_ln_g'], p['te_ln_b'])
    i_p = jnp.einsum('nh,hk->nk', Xc[..., 0], p['te_lt'][0])
    a_p = jnp.einsum('nhc,hk->nkc', Xc[..., 1:4], p['te_lt'][1])
    s_p = jnp.einsum('nhc,hk->nkc', Xc[..., 4:9], p['te_lt'][2])
    norm = _silu(norm @ p['te_ls1_w'] + p['te_ls1_b'])
    norm = _silu(norm @ p['te_ls2_w'] + p['te_ls2_b'])
    norm = norm.reshape(n, H, 3)
    Xc = jnp.concatenate([
        (i_p * norm[..., 0])[..., None],
        a_p * norm[..., 1][..., None],
        s_p * norm[..., 2][..., None],
    ], -1)

    perm = np.array([h * 3 + g for g in range(3) for h in range(H)])
    for l in range(NUM_LAYERS):
        ea = _silu(edge_attr @ p['int_ls1_w'][l] + p['int_ls1_b'][l])
        ea = _silu(ea @ p['int_ls2_w'][l] + p['int_ls2_b'][l])
        ea = _silu(ea @ p['int_ls3_w'][l][:, perm] + p['int_ls3_b'][l][perm])
        ea = ea * C[:, None]
        ea = jnp.pad(ea, ((0, epad), (0, CW - 192)))
        Xc = Xc / (_comps_norm(Xc) + 1.0)[..., None]
        i_p = jnp.einsum('nh,hk->nk', Xc[..., 0], p['int_lt'][l][0])
        a_p = jnp.einsum('nhc,hk->nck', Xc[..., 1:4], p['int_lt'][l][1])
        s_p = jnp.einsum('nhc,hk->nck', Xc[..., 4:9], p['int_lt'][l][2])
        T9 = jnp.concatenate([i_p[:, None, :], a_p, s_p], axis=1)
        Yc = jnp.swapaxes(T9, 1, 2)
        T9 = jnp.pad(T9.reshape(n, ROW), ((0, NPAD - n), (0, ROWP - ROW)))
        macc = _mp_inter_sc(T9, srcl, ldst, ea, bnd)
        Mc = jnp.swapaxes(macc[:n, :ROW].reshape(n, 9, H), 1, 2)
        Mf = _comps_to_full(Mc)
        Yf = _comps_to_full(Yc)
        P = jnp.matmul(Mf, Yf) + jnp.matmul(Yf, Mf)
        Pc = _full_to_comps(P)
        Pc = Pc / (_comps_norm(Pc) + 1.0)[..., None]
        i_p = jnp.einsum('nh,hk->nk', Pc[..., 0], p['int_lt'][l][3])
        a_p = jnp.einsum('nhc,hk->nkc', Pc[..., 1:4], p['int_lt'][l][4])
        s_p = jnp.einsum('nhc,hk->nkc', Pc[..., 4:9], p['int_lt'][l][5])
        dXc = jnp.concatenate([i_p[..., None], a_p, s_p], -1)
        dXf = _comps_to_full(dXc)
        dX2 = _full_to_comps(jnp.matmul(dXf, dXf))
        Xc = Xc + dXc + dX2

    return _head(Xc, p['out_ln_g'], p['out_ln_b'], p['lin_w'], p['lin_b'])


# speedup vs baseline: 1.0526x; 1.0526x over previous
import functools

import jax
import jax.numpy as jnp
import numpy as np
from jax import lax
from jax.experimental import pallas as pl
from jax.experimental.pallas import tpu as pltpu
from jax.experimental.pallas import tpu_sc as plsc

H = 64
NUM_RBF = 32
CUT_HI = 4.5
NUM_LAYERS = 2
PI = float(np.pi)


def _silu(x):
    return x * jax.nn.sigmoid(x)


def _cutoff(d):
    return 0.5 * (jnp.cos(d * (PI / CUT_HI)) + 1.0) * (d < CUT_HI)


def _expnorm_rbf(d):
    alpha = 5.0 / CUT_HI
    start = float(np.exp(-CUT_HI))
    means = jnp.linspace(start, 1.0, NUM_RBF).astype(jnp.float32)
    betas = (2.0 / NUM_RBF * (1.0 - start)) ** -2
    d = d[:, None]
    return _cutoff(d) * jnp.exp(-betas * (jnp.exp(-alpha * d) - means) ** 2)


def _layer_norm(x, g, b):
    m = x.mean(-1, keepdims=True)
    v = ((x - m) ** 2).mean(-1, keepdims=True)
    return (x - m) / jnp.sqrt(v + 1e-5) * g + b


def _norm9(cs):
    i, a0, a1, a2, s00, s01, s02, s11, s12 = cs
    s22 = -(s00 + s11)
    sn = s00**2 + s11**2 + s22**2 + 2.0 * (s01**2 + s02**2 + s12**2)
    return 3.0 * i**2 + 2.0 * (a0**2 + a1**2 + a2**2) + sn


def _to_full(cs):
    i, a0, a1, a2, s00, s01, s02, s11, s12 = cs
    s22 = -(s00 + s11)
    r0 = jnp.stack([i + s00, s01 - a2, s02 + a1], -1)
    r1 = jnp.stack([s01 + a2, i + s11, s12 - a0], -1)
    r2 = jnp.stack([s02 - a1, s12 + a0, i + s22], -1)
    return jnp.stack([r0, r1, r2], -2)


def _from_full(T):
    i = (T[..., 0, 0] + T[..., 1, 1] + T[..., 2, 2]) / 3.0
    a0 = 0.5 * (T[..., 2, 1] - T[..., 1, 2])
    a1 = 0.5 * (T[..., 0, 2] - T[..., 2, 0])
    a2 = 0.5 * (T[..., 1, 0] - T[..., 0, 1])
    s00 = T[..., 0, 0] - i
    s01 = 0.5 * (T[..., 0, 1] + T[..., 1, 0])
    s02 = 0.5 * (T[..., 0, 2] + T[..., 2, 0])
    s11 = T[..., 1, 1] - i
    s12 = 0.5 * (T[..., 1, 2] + T[..., 2, 1])
    return [i, a0, a1, a2, s00, s01, s02, s11, s12]


def _chan_lin(cs, W0, W1, W2):
    out = [cs[0] @ W0]
    out += [c @ W1 for c in cs[1:4]]
    out += [c @ W2 for c in cs[4:9]]
    return out


NC_T = 64
CHUNKS = 160
CPT = CHUNKS // 32
NPAD = NC_T * CHUNKS
EB = 64
ROW = 9 * H
ROWP = 640
CW = 256
_G9 = (0, 1, 1, 1, 2, 2, 2, 2, 2)


def _lane_select(vec, lane, fill):
    sel = lax.iota(jnp.int32, 16) == jnp.full((16,), lane, jnp.int32)
    return jnp.max(jnp.where(sel, vec, jnp.full((16,), fill, vec.dtype)))


def _chunk_bounds(bnd_hbm, bndb, ch):
    sl0 = pl.multiple_of(jnp.bitwise_and(ch, jnp.int32(-8)), 8)
    pltpu.sync_copy(bnd_hbm.at[pl.ds(sl0, 16)], bndb)
    lane = ch - sl0
    s_true = _lane_select(bndb[pl.ds(0, 16)], lane, 0)
    e_true = _lane_select(bndb[pl.ds(0, 16)], lane + 1, 0)
    s_al = jnp.bitwise_and(s_true, jnp.int32(-8))
    nb = lax.div(e_true - s_al + jnp.int32(EB - 1), jnp.int32(EB))
    return s_true, e_true, s_al, nb


def _zero_acc(acc):
    zf = jnp.zeros((16,), jnp.float32)

    def row_body(r, _):
        for v in range(ROWP // 16):
            acc[r, pl.ds(v * 16, 16)] = zf
        return 0

    lax.fori_loop(0, NC_T, row_body, 0)


def _valid_f(gid, s_true, e_true):
    ok = jnp.logical_and(gid >= s_true, gid < e_true)
    okv = jnp.full((16,), ok)
    return jnp.where(okv, jnp.full((16,), 1.0, jnp.float32),
                     jnp.zeros((16,), jnp.float32))


def _local_row(ldstb, j):
    base = pl.multiple_of(jnp.bitwise_and(j, jnp.int32(-16)), 16)
    return _lane_select(ldstb[pl.ds(base, 16)], jnp.bitwise_and(j, 15), 0)


def _mp_inter_body(T_hbm, srcl_hbm, ldst_hbm, coef_hbm, bnd_hbm, out_hbm,
                   acc, idxb, ldstb, coefb, rowsb, bndb, sem):
    w = lax.axis_index("s") * 2 + lax.axis_index("c")
    for q in range(CPT):
        ch = w * CPT + q
        _zero_acc(acc)
        s_true, e_true, s_al, nb = _chunk_bounds(bnd_hbm, bndb, ch)

        def batch_body(i, _):
            sb = pl.multiple_of(s_al + i * EB, 8)
            pltpu.sync_copy(srcl_hbm.at[pl.ds(sb, EB)], idxb)
            pltpu.sync_copy(ldst_hbm.at[pl.ds(sb, EB)], ldstb)
            pltpu.sync_copy(coef_hbm.at[pl.ds(sb, EB)], coefb)
            pltpu.async_copy(T_hbm.at[idxb], rowsb, sem).wait()

            def edge_body(j, _2):
                okf = _valid_f(sb + j, s_true, e_true)
                ld = _local_row(ldstb, j)
                cvs = [coefb[j, pl.ds(wi * 16, 16)] * okf for wi in range(12)]
                for v in range(36):
                    g = _G9[v // 4]
                    sl = pl.ds(v * 16, 16)
                    acc[ld, sl] = acc[ld, sl] + rowsb[j, sl] * cvs[g * 4 + (v % 4)]
                return 0

            lax.fori_loop(0, EB, edge_body, 0)
            return 0

        lax.fori_loop(0, nb, batch_body, 0)
        pltpu.sync_copy(acc, out_hbm.at[pl.ds(ch * NC_T, NC_T)])


def _mp_embed_body(pz_hbm, dstg_hbm, srcl_hbm, ldst_hbm, w_hbm, bnd_hbm,
                   out_hbm,
                   acc, dgb, idxb, ldstb, pab, pbb, wb, bndb, sem):
    w = lax.axis_index("s") * 2 + lax.axis_index("c")
    for q in range(CPT):
        ch = w * CPT + q
        _zero_acc(acc)
        s_true, e_true, s_al, nb = _chunk_bounds(bnd_hbm, bndb, ch)

        def batch_body(i, _):
            sb = pl.multiple_of(s_al + i * EB, 8)
            pltpu.sync_copy(dstg_hbm.at[pl.ds(sb, EB)], dgb)
            pltpu.sync_copy(srcl_hbm.at[pl.ds(sb, EB)], idxb)
            pltpu.sync_copy(ldst_hbm.at[pl.ds(sb, EB)], ldstb)
            pltpu.sync_copy(w_hbm.at[pl.ds(sb, EB)], wb)
            cp_a = pltpu.async_copy(pz_hbm.at[dgb], pab, sem)
            cp_b = pltpu.async_copy(pz_hbm.at[idxb], pbb, sem)
            cp_a.wait()
            cp_b.wait()

            def edge_body(j, _2):
                okf = _valid_f(sb + j, s_true, e_true)
                ld = _local_row(ldstb, j)
                zv = [pab[j, pl.ds(wi * 16, 16)] + pbb[j, pl.ds(64 + wi * 16, 16)]
                      for wi in range(4)]
                cvs = [zv[wi % 4] * wb[j, pl.ds(wi * 16, 16)] * okf
                       for wi in range(12)]
                gvec = wb[j, pl.ds(192, 16)]
                gvs = [_lane_select(gvec, cc, jnp.float32(-3e38))
                       for cc in range(9)]
                for v in range(36):
                    cc = v // 4
                    g = _G9[cc]
                    sl = pl.ds(v * 16, 16)
                    acc[ld, sl] = acc[ld, sl] + cvs[g * 4 + (v % 4)] * gvs[cc]
                return 0

            lax.fori_loop(0, EB, edge_body, 0)
            return 0

        lax.fori_loop(0, nb, batch_body, 0)
        pltpu.sync_copy(acc, out_hbm.at[pl.ds(ch * NC_T, NC_T)])


_SC_MESH = dict(
    mesh=plsc.VectorSubcoreMesh(core_axis_name="c", subcore_axis_name="s"),
    compiler_params=pltpu.CompilerParams(needs_layout_passes=False),
)


def _mp_inter_sc(T9, srcl, ldst, coef, bnd):
    f = pl.kernel(
        _mp_inter_body,
        out_type=jax.ShapeDtypeStruct((NPAD, ROWP), jnp.float32),
        scratch_types=[
            pltpu.VMEM((NC_T, ROWP), jnp.float32),
            pltpu.VMEM((EB,), jnp.int32),
            pltpu.VMEM((EB,), jnp.int32),
            pltpu.VMEM((EB, CW), jnp.float32),
            pltpu.VMEM((EB, ROWP), jnp.float32),
            pltpu.VMEM((16,), jnp.int32),
            pltpu.SemaphoreType.DMA,
        ],
        **_SC_MESH,
    )
    return f(T9, srcl, ldst, coef, bnd)


def _mp_embed_sc(pz, dstg, srcl, ldst, w123, bnd):
    f = pl.kernel(
        _mp_embed_body,
        out_type=jax.ShapeDtypeStruct((NPAD, ROWP), jnp.float32),
        scratch_types=[
            pltpu.VMEM((NC_T, ROWP), jnp.float32),
            pltpu.VMEM((EB,), jnp.int32),
            pltpu.VMEM((EB,), jnp.int32),
            pltpu.VMEM((EB,), jnp.int32),
            pltpu.VMEM((EB, 128), jnp.float32),
            pltpu.VMEM((EB, 128), jnp.float32),
            pltpu.VMEM((EB, CW), jnp.float32),
            pltpu.VMEM((16,), jnp.int32),
            pltpu.SemaphoreType.DMA,
        ],
        **_SC_MESH,
    )
    return f(pz, dstg, srcl, ldst, w123, bnd)


_NB = 512


def _head_body(x_ref, g_ref, b_ref, w_ref, lb_ref, o_ref):
    x = x_ref[...]
    i = x[:, 0:64]
    a0, a1, a2 = x[:, 64:128], x[:, 128:192], x[:, 192:256]
    s00, s01, s02 = x[:, 256:320], x[:, 320:384], x[:, 384:448]
    s11, s12 = x[:, 448:512], x[:, 512:576]
    s22 = -(s00 + s11)
    nI = 3.0 * i * i
    nA = 2.0 * (a0 * a0 + a1 * a1 + a2 * a2)
    nS = s00 * s00 + s11 * s11 + s22 * s22 + 2.0 * (s01 * s01 + s02 * s02 + s12 * s12)
    v = jnp.concatenate([nI, nA, nS], axis=1)
    m = v.mean(-1, keepdims=True)
    var = ((v - m) ** 2).mean(-1, keepdims=True)
    v = (v - m) / jnp.sqrt(var + 1e-5) * g_ref[...] + b_ref[...]
    y = jnp.dot(v, w_ref[...], preferred_element_type=jnp.float32) + lb_ref[...]
    o_ref[...] = y * jax.nn.sigmoid(y)


def _head(x, g, b, w, lb):
    npad = x.shape[0]
    out = pl.pallas_call(
        _head_body,
        grid=(npad // _NB,),
        in_specs=[
            pl.BlockSpec((_NB, 9 * H), lambda i: (i, 0)),
            pl.BlockSpec((3 * H,), lambda i: (0,)),
            pl.BlockSpec((3 * H,), lambda i: (0,)),
            pl.BlockSpec((3 * H, H), lambda i: (0, 0)),
            pl.BlockSpec((H,), lambda i: (0,)),
        ],
        out_specs=pl.BlockSpec((_NB, H), lambda i: (i, 0)),
        out_shape=jax.ShapeDtypeStruct((npad, H), jnp.float32),
    )(x, g, b, w, lb)
    return out


def kernel(z, edge_index, edge_weight, edge_vec, params):
    p = params
    n = z.shape[0]
    E = edge_index.shape[1]
    EPAD = E + 2 * EB
    epad = EPAD - E

    dst0 = edge_index[0].astype(jnp.int32)
    order = jnp.argsort(dst0)
    dst = dst0[order]
    src = edge_index[1].astype(jnp.int32)[order]
    ew = jnp.pad(edge_weight[order], (0, epad), constant_values=2.0 * CUT_HI)
    ev = jnp.pad(edge_vec[order], ((0, epad), (0, 0)), constant_values=1.0)

    b = jnp.searchsorted(dst, jnp.arange(0, NPAD + 1, NC_T,
                                         dtype=jnp.int32)).astype(jnp.int32)
    bnd = jnp.pad(b, (0, 176 - (CHUNKS + 1)), mode='edge')
    srcl = jnp.pad(src, (0, epad))
    dstg = jnp.pad(dst, (0, epad))
    ldst = jnp.bitwise_and(dstg, NC_T - 1)

    edge_attr = _expnorm_rbf(ew)
    C = _cutoff(ew)
    evn = ev / ew[:, None]

    Wd = jnp.concatenate([p['dproj_w'][0], p['dproj_w'][1],
                          p['dproj_w'][2]], axis=1)
    bd = jnp.concatenate([p['dproj_b'][0], p['dproj_b'][1], p['dproj_b'][2]])
    v0, v1, v2 = evn[:, 0], evn[:, 1], evn[:, 2]
    q = (v0 * v0 + v1 * v1 + v2 * v2) / 3.0
    gcC = jnp.stack([jnp.ones_like(v0), v0, v1, v2,
                     v0 * v0 - q, v0 * v1, v0 * v2, v1 * v1 - q,
                     v1 * v2], -1) * C[:, None]
    F = jnp.concatenate([edge_attr, gcC, jnp.ones((EPAD, 1), jnp.float32)], 1)
    Wbig = jnp.zeros((42, CW), jnp.float32)
    Wbig = Wbig.at[0:32, 0:192].set(Wd)
    Wbig = Wbig.at[32:41, 192:201].set(jnp.eye(9, dtype=jnp.float32))
    Wbig = Wbig.at[41, 0:192].set(bd)
    Wcat = F @ Wbig
    Z = jnp.take(p['emb'], z, axis=0)
    PZ = jnp.concatenate([Z @ p['emb2_w'][:H] + p['emb2_b'],
                          Z @ p['emb2_w'][H:]], axis=1)
    acc = _mp_embed_sc(PZ, dstg, srcl, ldst, Wcat, bnd)
    A = acc.reshape(NPAD, 10, H)
    cs = [A[:, k] for k in range(9)]

    norm = _layer_norm(_norm9(cs), p['te_ln_g'], p['te_ln_b'])
    cs = _chan_lin(cs, p['te_lt'][0], p['te_lt'][1], p['te_lt'][2])
    norm = _silu(norm @ p['te_ls1_w'] + p['te_ls1_b'])
    norm = _silu(norm @ p['te_ls2_w'] + p['te_ls2_b'])
    norm = norm.reshape(NPAD, H, 3)
    n0, n1, n2 = norm[..., 0], norm[..., 1], norm[..., 2]
    cs = ([cs[0] * n0] + [c * n1 for c in cs[1:4]]
          + [c * n2 for c in cs[4:9]])

    perm = np.array([h * 3 + g for g in range(3) for h in range(H)])
    zpad = jnp.zeros((NPAD, 1, H), jnp.float32)
    for l in range(NUM_LAYERS):
        W3p = jnp.zeros((2 * H, CW), jnp.float32)
        W3p = W3p.at[:, 0:192].set(p['int_ls3_w'][l][:, perm])
        b3p = jnp.zeros((CW,), jnp.float32).at[0:192].set(
            p['int_ls3_b'][l][perm])
        ea = _silu(edge_attr @ p['int_ls1_w'][l] + p['int_ls1_b'][l])
        ea = _silu(ea @ p['int_ls2_w'][l] + p['int_ls2_b'][l])
        ea = _silu(ea @ W3p + b3p) * C[:, None]
        nrm = (_norm9(cs) + 1.0)
        cs = [c / nrm for c in cs]
        ys = _chan_lin(cs, p['int_lt'][l][0], p['int_lt'][l][1],
                       p['int_lt'][l][2])
        T9 = jnp.concatenate([jnp.stack(ys, axis=1), zpad],
                             axis=1).reshape(NPAD, ROWP)
        macc = _mp_inter_sc(T9, srcl, ldst, ea, bnd)
        M = macc.reshape(NPAD, 10, H)
        ms = [M[:, k] for k in range(9)]
        Mf = _to_full(ms)
        Yf = _to_full(ys)
        P = jnp.matmul(Mf, Yf) + jnp.matmul(Yf, Mf)
        ps = _from_full(P)
        nrm = (_norm9(ps) + 1.0)
        ps = [c / nrm for c in ps]
        ds = _chan_lin(ps, p['int_lt'][l][3], p['int_lt'][l][4],
                       p['int_lt'][l][5])
        d2 = _from_full(jnp.matmul(_to_full(ds), _to_full(ds)))
        cs = [x + d + e for x, d, e in zip(cs, ds, d2)]

    x576 = jnp.stack(cs, axis=1).reshape(NPAD, ROW)
    out = _head(x576, p['out_ln_g'], p['out_ln_b'], p['lin_w'], p['lin_b'])
    return out[:n]
